# baseline (speedup 1.0000x reference)
import functools

import jax
import jax.numpy as jnp
from jax import lax
from jax.experimental import pallas as pl
from jax.experimental.pallas import tpu as pltpu
from jax.experimental.pallas import tpu_sc as plsc

B = 64
P = 1024
MAX_PATCH = 32
EMB = 64
PATCH_DIM = 256
FLAT = EMB * MAX_PATCH
EMBW = EMB // 2

N = B * P
L = N * MAX_PATCH

_info = plsc.get_sparse_core_info()
_NC = _info.num_cores
_NS = _info.num_subcores
NW = _NC * _NS
CHUNK = 2048
NCHUNK = N // CHUNK

_sc_mesh = plsc.VectorSubcoreMesh(core_axis_name="c", subcore_axis_name="s")


@functools.partial(
    pl.kernel,
    mesh=_sc_mesh,
    out_type=jax.ShapeDtypeStruct((MAX_PATCH, N, EMBW), jnp.int32),
    scratch_types=[
        pltpu.VMEM((CHUNK,), jnp.int32),
        pltpu.VMEM((CHUNK, EMBW), jnp.int32),
        pltpu.VMEM_SHARED((256, EMBW), jnp.int32),
        pltpu.SemaphoreType.DMA,
    ],
    compiler_params=pltpu.CompilerParams(use_tc_tiling_on_sc=False),
)
def _sc_gather(table_hbm, idx_hbm, out_hbm, idx_v, rows_v, tab_sp, sem):
    sid = lax.axis_index("s")
    wid = sid * _NC + lax.axis_index("c")

    @pl.when(sid == 0)
    def _():
        pltpu.sync_copy(table_hbm, tab_sp)
    plsc.subcore_barrier()

    def body(g, carry):
        off = g * CHUNK
        pltpu.sync_copy(idx_hbm.at[wid, pl.ds(off, CHUNK)], idx_v)
        pltpu.async_copy(tab_sp.at[idx_v], rows_v, sem).wait()
        pltpu.sync_copy(rows_v, out_hbm.at[wid, pl.ds(off, CHUNK)])
        return carry

    lax.fori_loop(0, NCHUNK, body, 0)


def _tc_body(emb_ref, W1_ref, b1_ref, W2_ref, b2_ref, gamma_ref, beta_ref,
             out_ref):
    acc = jnp.zeros((P, PATCH_DIM), jnp.float32)
    for j in range(MAX_PATCH):
        xw = emb_ref[j]
        lo = jax.lax.bitcast_convert_type(xw << 16, jnp.float32)
        hi = jax.lax.bitcast_convert_type(
            xw & jnp.int32(-65536), jnp.float32)
        x = jnp.concatenate([lo, hi], axis=1)
        acc = acc + jnp.dot(x.astype(jnp.bfloat16), W1_ref[j],
                            preferred_element_type=jnp.float32)

    h = acc + b1_ref[0]
    h = h * 0.5 * (1.0 + jax.lax.erf(h * 0.7071067811865476))
    h = jnp.dot(h.astype(jnp.bfloat16), W2_ref[...],
                preferred_element_type=jnp.float32) + b2_ref[0]

    mu = jnp.mean(h, axis=1, keepdims=True)
    var = jnp.mean(jnp.square(h - mu), axis=1, keepdims=True)
    y = (h - mu) * jax.lax.rsqrt(var + 1e-5)
    y = y * gamma_ref[0] + beta_ref[0]

    out_ref[0, 0, :] = jnp.mean(y, axis=0)


def kernel(byte_ids, table, W1, b1, W2, b2, gamma, beta):
    table_i32 = jax.lax.bitcast_convert_type(
        table.astype(jnp.bfloat16).reshape(256, EMBW, 2), jnp.int32)
    idsT = byte_ids.reshape(N, MAX_PATCH).T
    perm = jnp.concatenate([jnp.arange(0, EMB, 2), jnp.arange(1, EMB, 2)])
    W1_bf = W1.reshape(MAX_PATCH, EMB, PATCH_DIM)[:, perm, :].astype(
        jnp.bfloat16)
    W2_bf = W2.astype(jnp.bfloat16)
    b1r = b1.reshape(1, PATCH_DIM)
    b2r = b2.reshape(1, PATCH_DIM)
    gammar = gamma.reshape(1, PATCH_DIM)
    betar = beta.reshape(1, PATCH_DIM)

    emb = _sc_gather(table_i32, idsT)

    out = pl.pallas_call(
        _tc_body,
        grid=(B,),
        in_specs=[
            pl.BlockSpec((MAX_PATCH, P, EMBW), lambda b: (0, b, 0)),
            pl.BlockSpec((MAX_PATCH, EMB, PATCH_DIM), lambda b: (0, 0, 0)),
            pl.BlockSpec((1, PATCH_DIM), lambda b: (0, 0)),
            pl.BlockSpec((PATCH_DIM, PATCH_DIM), lambda b: (0, 0)),
            pl.BlockSpec((1, PATCH_DIM), lambda b: (0, 0)),
            pl.BlockSpec((1, PATCH_DIM), lambda b: (0, 0)),
            pl.BlockSpec((1, PATCH_DIM), lambda b: (0, 0)),
        ],
        out_specs=pl.BlockSpec((1, 1, PATCH_DIM), lambda b: (b, 0, 0)),
        out_shape=jax.ShapeDtypeStruct((B, 1, PATCH_DIM), jnp.float32),
        compiler_params=pltpu.CompilerParams(
            dimension_semantics=("arbitrary",),
        ),
    )(emb, W1_bf, b1r, W2_bf, b2r, gammar, betar)
    return out.reshape(B, PATCH_DIM)


# speedup vs baseline: 3.4771x; 3.4771x over previous
import jax
import jax.numpy as jnp
from jax.experimental import pallas as pl
from jax.experimental.pallas import tpu as pltpu

B = 64
P = 1024
MAX_PATCH = 32
EMB = 64
PATCH_DIM = 256
FLAT = EMB * MAX_PATCH
GB = 2
M = GB * P


def _body(ids_ref, table_ref, W1_ref, b1_ref, W2_ref, b2_ref, gamma_ref,
          beta_ref, out_ref, bigT_ref, oh_ref):
    @pl.when(pl.program_id(0) == 0)
    def _():
        tab = table_ref[...]
        for j in range(MAX_PATCH):
            w1j = W1_ref[pl.ds(j * EMB, EMB), :]
            bigT_ref[pl.ds(j * 256, 256), :] = jnp.dot(
                tab, w1j, preferred_element_type=jnp.float32
            ).astype(jnp.bfloat16)

    ids16 = ids_ref[...].reshape(M, MAX_PATCH).astype(jnp.int16)
    iota16 = jax.lax.broadcasted_iota(jnp.int16, (M, PATCH_DIM), 1)

    for j in range(MAX_PATCH):
        col = ids16[:, j:j + 1]
        oh = jnp.where(col == iota16, jnp.bfloat16(1), jnp.bfloat16(0))
        oh_ref[:, pl.ds(j * 256, 256)] = oh

    h = jnp.dot(oh_ref[...], bigT_ref[...],
                preferred_element_type=jnp.float32)

    h = h + b1_ref[0]
    h = h * 0.5 * (1.0 + jax.lax.erf(h * 0.7071067811865476))
    h = jnp.dot(h.astype(jnp.bfloat16), W2_ref[...].astype(jnp.bfloat16),
                preferred_element_type=jnp.float32) + b2_ref[0]

    mu = jnp.mean(h, axis=1, keepdims=True)
    var = jnp.mean(jnp.square(h - mu), axis=1, keepdims=True)
    y = (h - mu) * jax.lax.rsqrt(var + 1e-5)
    y = y * gamma_ref[0] + beta_ref[0]

    out_ref[...] = jnp.mean(y.reshape(GB, P, PATCH_DIM), axis=1,
                            keepdims=True)


def kernel(byte_ids, table, W1, b1, W2, b2, gamma, beta):
    ids3 = byte_ids.reshape(B, P, MAX_PATCH)
    b1r = b1.reshape(1, PATCH_DIM)
    b2r = b2.reshape(1, PATCH_DIM)
    gammar = gamma.reshape(1, PATCH_DIM)
    betar = beta.reshape(1, PATCH_DIM)

    grid = (B // GB,)
    out = pl.pallas_call(
        _body,
        grid=grid,
        in_specs=[
            pl.BlockSpec((GB, P, MAX_PATCH), lambda b: (b, 0, 0)),
            pl.BlockSpec((256, EMB), lambda b: (0, 0)),
            pl.BlockSpec((FLAT, PATCH_DIM), lambda b: (0, 0)),
            pl.BlockSpec((1, PATCH_DIM), lambda b: (0, 0)),
            pl.BlockSpec((PATCH_DIM, PATCH_DIM), lambda b: (0, 0)),
            pl.BlockSpec((1, PATCH_DIM), lambda b: (0, 0)),
            pl.BlockSpec((1, PATCH_DIM), lambda b: (0, 0)),
            pl.BlockSpec((1, PATCH_DIM), lambda b: (0, 0)),
        ],
        out_specs=pl.BlockSpec((GB, 1, PATCH_DIM), lambda b: (b, 0, 0)),
        out_shape=jax.ShapeDtypeStruct((B, 1, PATCH_DIM), jnp.float32),
        scratch_shapes=[
            pltpu.VMEM((MAX_PATCH * 256, PATCH_DIM), jnp.bfloat16),
            pltpu.VMEM((M, MAX_PATCH * 256), jnp.bfloat16),
        ],
        compiler_params=pltpu.CompilerParams(
            dimension_semantics=("arbitrary",),
        ),
    )(ids3, table, W1, b1r, W2, b2r, gammar, betar)
    return out.reshape(B, PATCH_DIM)
